# baseline (speedup 1.0000x reference)
import functools

import jax
import jax.numpy as jnp
import numpy as np
from jax import lax
from jax.experimental import pallas as pl
from jax.experimental.pallas import tpu as pltpu
from jax.experimental.pallas import tpu_sc as plsc

_PATCH = 14
_NC, _NS = 2, 16
_NW = _NC * _NS
_LANES = 16
_CHUNKS = 8

_TABLE_CACHE = {}


def _perm_tables(B, nh, nw):
    key_ = (B, nh, nw)
    p = _PATCH
    L = nh * nw
    rows = nh * p * nw

    def make():
        key = jax.random.key(42)
        keys = jax.random.split(key, B)
        return jnp.stack([jax.random.permutation(k, L) for k in keys])

    if key_ not in _TABLE_CACHE:
        try:
            with jax.default_device(jax.devices("cpu")[0]), \
                    jax.ensure_compile_time_eval():
                _TABLE_CACHE[key_] = np.asarray(make())
        except Exception:
            pass

    if key_ in _TABLE_CACHE:
        perms = _TABLE_CACHE[key_]
        xp = np
    else:
        perms = make()
        xp = jnp

    sh = (perms // nw).reshape(B, nh, nw)
    sw = (perms % nw).reshape(B, nh, nw)
    r = xp.arange(p, dtype=xp.int32)
    off_h = sh[:, :, None, :] * p + r[None, None, :, None]
    off_w = xp.broadcast_to(sw[:, :, None, :] * p, (B, nh, p, nw))
    return (off_h.reshape(B, rows).astype(xp.int32),
            off_w.reshape(B, rows).astype(xp.int32))


def kernel(x):
    B, C, H, W = x.shape
    p = _PATCH
    nh, nw = H // p, W // p
    rows = H * nw
    n_img = B * C
    assert n_img % _NW == 0
    imgs_per_w = n_img // _NW
    assert imgs_per_w % 2 == 0
    assert C % imgs_per_w == 0
    assert H % _CHUNKS == 0

    chunk_h = H // _CHUNKS
    blk_per_chunk = chunk_h

    off_h_np, off_w_np = _perm_tables(B, nh, nw)
    off_h = jnp.asarray(off_h_np)
    off_w = jnp.asarray(off_w_np)

    def body(x_ref, offh_ref, offw_ref, out_ref,
             in0, in1, st0, st1, offh_v, offw_v, si0, si1, ss0, ss1):
        cid = lax.axis_index("c")
        sid = lax.axis_index("s")
        wid = sid * _NC + cid
        first = wid * imgs_per_w
        b = first // C
        c0 = first - b * C
        pltpu.sync_copy(offh_ref.at[b], offh_v)
        pltpu.sync_copy(offw_ref.at[b], offw_v)

        iota = lax.iota(jnp.int32, _LANES)
        i14 = iota * jnp.full((_LANES,), p, jnp.int32)
        one = jnp.full((_LANES,), 1, jnp.int32)
        cvecs = [i14 + jnp.full((_LANES,), j, jnp.int32) for j in range(p)]

        ins = [in0, in1]
        isems = [si0, si1]
        stages = [st0, st1]
        ssems = [ss0, ss1]

        def shuffle_chunk(inbuf, stage, c):
            def blk(tt, carry):
                o0 = (c * chunk_h + tt) * nw
                hvec = offh_v[pl.ds(o0, _LANES)]
                wvec = offw_v[pl.ds(o0, _LANES)]
                row = stage.at[tt]
                for j in range(p):
                    v = plsc.load_gather(inbuf, [hvec, wvec])
                    plsc.store_scatter(row, [cvecs[j]], v)
                    wvec = wvec + one
                return carry

            lax.fori_loop(0, blk_per_chunk, blk, 0)

        pltpu.async_copy(x_ref.at[b, c0], in0, si0)
        pltpu.async_copy(x_ref.at[b, c0 + 1], in1, si1)

        def pair(k2, carry):
            for h in range(2):
                k = k2 * 2 + h
                cc = c0 + k
                pltpu.make_async_copy(
                    x_ref.at[b, cc], ins[h], isems[h]).wait()
                for c in range(_CHUNKS):
                    st = stages[c % 2]
                    if c >= 2:
                        pltpu.make_async_copy(
                            st,
                            out_ref.at[b, cc, pl.ds((c - 2) * chunk_h,
                                                    chunk_h)],
                            ssems[c % 2]).wait()
                    shuffle_chunk(ins[h], st, c)
                    pltpu.async_copy(
                        st,
                        out_ref.at[b, cc, pl.ds(c * chunk_h, chunk_h)],
                        ssems[c % 2])
                @pl.when(k2 + 1 < imgs_per_w // 2)
                def _():
                    pltpu.async_copy(x_ref.at[b, cc + 2], ins[h], isems[h])
                for c in (_CHUNKS - 2, _CHUNKS - 1):
                    pltpu.make_async_copy(
                        stages[c % 2],
                        out_ref.at[b, cc, pl.ds(c * chunk_h, chunk_h)],
                        ssems[c % 2]).wait()
            return carry

        lax.fori_loop(0, imgs_per_w // 2, pair, 0)

    f = pl.kernel(
        body,
        out_type=jax.ShapeDtypeStruct((B, C, H, W), jnp.float32),
        mesh=plsc.VectorSubcoreMesh(
            core_axis_name="c", subcore_axis_name="s",
            num_cores=_NC, num_subcores=_NS),
        compiler_params=pltpu.CompilerParams(
            use_tc_tiling_on_sc=False, needs_layout_passes=False),
        scratch_types=[
            pltpu.VMEM((H, W), jnp.float32),
            pltpu.VMEM((H, W), jnp.float32),
            pltpu.VMEM((chunk_h, W), jnp.float32),
            pltpu.VMEM((chunk_h, W), jnp.float32),
            pltpu.VMEM((rows,), jnp.int32),
            pltpu.VMEM((rows,), jnp.int32),
            pltpu.SemaphoreType.DMA,
            pltpu.SemaphoreType.DMA,
            pltpu.SemaphoreType.DMA,
            pltpu.SemaphoreType.DMA,
        ],
    )
    return f(x, off_h, off_w)


# speedup vs baseline: 1.8675x; 1.8675x over previous
import functools

import jax
import jax.numpy as jnp
import numpy as np
from jax import lax
from jax.experimental import pallas as pl
from jax.experimental.pallas import tpu as pltpu
from jax.experimental.pallas import tpu_sc as plsc

_PATCH = 14
_NC, _NS = 2, 16
_NW = _NC * _NS
_LANES = 16
_CHUNKS = 14

_TABLE_CACHE = {}


def _perm_tables(B, nh, nw):
    key_ = (B, nh, nw)
    p = _PATCH
    L = nh * nw
    rows = nh * p * nw

    def make():
        key = jax.random.key(42)
        keys = jax.random.split(key, B)
        return jnp.stack([jax.random.permutation(k, L) for k in keys])

    if key_ not in _TABLE_CACHE:
        try:
            with jax.default_device(jax.devices("cpu")[0]), \
                    jax.ensure_compile_time_eval():
                _TABLE_CACHE[key_] = np.asarray(make())
        except Exception:
            pass

    if key_ in _TABLE_CACHE:
        perms = _TABLE_CACHE[key_]
        xp = np
    else:
        perms = make()
        xp = jnp

    sh = (perms // nw).reshape(B, nh, nw)
    sw = (perms % nw).reshape(B, nh, nw)
    r = xp.arange(p, dtype=xp.int32)
    off_h = sh[:, :, None, :] * p + r[None, None, :, None]
    off_w = xp.broadcast_to(sw[:, :, None, :] * p, (B, nh, p, nw))
    return (off_h.reshape(B, rows).astype(xp.int32),
            off_w.reshape(B, rows).astype(xp.int32))


def kernel(x):
    B, C, H, W = x.shape
    p = _PATCH
    nh, nw = H // p, W // p
    rows = H * nw
    n_img = B * C
    assert n_img % _NW == 0
    imgs_per_w = n_img // _NW
    assert imgs_per_w % 2 == 0
    assert C % imgs_per_w == 0
    assert H % _CHUNKS == 0

    chunk_h = H // _CHUNKS
    blk_per_chunk = chunk_h

    off_h_np, off_w_np = _perm_tables(B, nh, nw)
    off_h = jnp.asarray(off_h_np)
    off_w = jnp.asarray(off_w_np)

    def body(x_ref, offh_ref, offw_ref, out_ref,
             in0, in1, st0, st1, offh_v, offw_v, si0, si1, ss0, ss1):
        cid = lax.axis_index("c")
        sid = lax.axis_index("s")
        wid = sid * _NC + cid
        first = wid * imgs_per_w
        b = first // C
        c0 = first - b * C
        pltpu.sync_copy(offh_ref.at[b], offh_v)
        pltpu.sync_copy(offw_ref.at[b], offw_v)

        iota = lax.iota(jnp.int32, _LANES)
        i14 = iota * jnp.full((_LANES,), p, jnp.int32)
        one = jnp.full((_LANES,), 1, jnp.int32)
        cvecs = [i14 + jnp.full((_LANES,), j, jnp.int32) for j in range(p)]

        ins = [in0, in1]
        isems = [si0, si1]
        stages = [st0, st1]
        ssems = [ss0, ss1]

        def shuffle_chunk(inbuf, stage, c):
            def blk(tt, carry):
                o0 = (c * chunk_h + tt) * nw
                hvec = offh_v[pl.ds(o0, _LANES)]
                wvec = offw_v[pl.ds(o0, _LANES)]
                rvec = jnp.full((_LANES,), tt, jnp.int32)
                for j in range(p):
                    v = plsc.load_gather(inbuf, [hvec, wvec])
                    plsc.store_scatter(stage, [rvec, cvecs[j]], v)
                    wvec = wvec + one
                return carry

            lax.fori_loop(0, blk_per_chunk, blk, 0)

        pltpu.async_copy(x_ref.at[b, c0], in0, si0)
        pltpu.async_copy(x_ref.at[b, c0 + 1], in1, si1)

        def pair(k2, carry):
            for h in range(2):
                k = k2 * 2 + h
                cc = c0 + k
                pltpu.make_async_copy(
                    x_ref.at[b, cc], ins[h], isems[h]).wait()
                for c in range(_CHUNKS):
                    st = stages[c % 2]
                    if c >= 2:
                        pltpu.make_async_copy(
                            st,
                            out_ref.at[b, cc, pl.ds((c - 2) * chunk_h,
                                                    chunk_h)],
                            ssems[c % 2]).wait()
                    shuffle_chunk(ins[h], st, c)
                    pltpu.async_copy(
                        st,
                        out_ref.at[b, cc, pl.ds(c * chunk_h, chunk_h)],
                        ssems[c % 2])
                @pl.when(k2 + 1 < imgs_per_w // 2)
                def _():
                    pltpu.async_copy(x_ref.at[b, cc + 2], ins[h], isems[h])
                for c in (_CHUNKS - 2, _CHUNKS - 1):
                    pltpu.make_async_copy(
                        stages[c % 2],
                        out_ref.at[b, cc, pl.ds(c * chunk_h, chunk_h)],
                        ssems[c % 2]).wait()
            return carry

        lax.fori_loop(0, imgs_per_w // 2, pair, 0)

    f = pl.kernel(
        body,
        out_type=jax.ShapeDtypeStruct((B, C, H, W), jnp.float32),
        mesh=plsc.VectorSubcoreMesh(
            core_axis_name="c", subcore_axis_name="s",
            num_cores=_NC, num_subcores=_NS),
        compiler_params=pltpu.CompilerParams(
            use_tc_tiling_on_sc=True, needs_layout_passes=False),
        scratch_types=[
            pltpu.VMEM((H, W), jnp.float32),
            pltpu.VMEM((H, W), jnp.float32),
            pltpu.VMEM((chunk_h, W), jnp.float32),
            pltpu.VMEM((chunk_h, W), jnp.float32),
            pltpu.VMEM((rows,), jnp.int32),
            pltpu.VMEM((rows,), jnp.int32),
            pltpu.SemaphoreType.DMA,
            pltpu.SemaphoreType.DMA,
            pltpu.SemaphoreType.DMA,
            pltpu.SemaphoreType.DMA,
        ],
    )
    return f(x, off_h, off_w)


# speedup vs baseline: 3.4143x; 1.8283x over previous
import functools

import jax
import jax.numpy as jnp
import numpy as np
from jax import lax
from jax.experimental import pallas as pl
from jax.experimental.pallas import tpu as pltpu
from jax.experimental.pallas import tpu_sc as plsc

_PATCH = 14
_NC, _NS = 2, 16
_NW = _NC * _NS
_LANES = 16
_CHUNKS = 14

_TABLE_CACHE = {}


def _perm_tables(B, nh, nw):
    key_ = (B, nh, nw)
    p = _PATCH
    L = nh * nw
    rows = nh * p * nw

    def make():
        key = jax.random.key(42)
        keys = jax.random.split(key, B)
        return jnp.stack([jax.random.permutation(k, L) for k in keys])

    if key_ not in _TABLE_CACHE:
        try:
            with jax.default_device(jax.devices("cpu")[0]), \
                    jax.ensure_compile_time_eval():
                _TABLE_CACHE[key_] = np.asarray(make())
        except Exception:
            pass

    if key_ in _TABLE_CACHE:
        perms = _TABLE_CACHE[key_]
        xp = np
    else:
        perms = make()
        xp = jnp

    sh = (perms // nw).reshape(B, nh, nw)
    sw = (perms % nw).reshape(B, nh, nw)
    r = xp.arange(p, dtype=xp.int32)
    off_h = sh[:, :, None, :] * p + r[None, None, :, None]
    off_w = xp.broadcast_to(sw[:, :, None, :] * p, (B, nh, p, nw))
    return (off_h.reshape(B, rows).astype(xp.int32),
            off_w.reshape(B, rows).astype(xp.int32))


def kernel(x):
    B, C, H, W = x.shape
    p = _PATCH
    nh, nw = H // p, W // p
    rows = H * nw
    n_img = B * C
    assert n_img % _NW == 0
    imgs_per_w = n_img // _NW
    assert imgs_per_w % 2 == 0
    assert C % imgs_per_w == 0
    assert H % _CHUNKS == 0

    chunk_h = H // _CHUNKS
    blk_per_chunk = chunk_h

    off_h_np, off_w_np = _perm_tables(B, nh, nw)
    off_h = jnp.asarray(off_h_np)
    off_w = jnp.asarray(off_w_np)

    def body(x_ref, offh_ref, offw_ref, out_ref,
             in0, in1, st0, st1, offh_v, offw_v, si0, si1, ss0, ss1):
        cid = lax.axis_index("c")
        sid = lax.axis_index("s")
        wid = sid * _NC + cid
        first = wid * imgs_per_w
        b = first // C
        c0 = first - b * C
        pltpu.sync_copy(offh_ref.at[b], offh_v)
        pltpu.sync_copy(offw_ref.at[b], offw_v)

        iota = lax.iota(jnp.int32, _LANES)
        i14 = iota * jnp.full((_LANES,), p, jnp.int32)
        one = jnp.full((_LANES,), 1, jnp.int32)
        cvecs = [i14 + jnp.full((_LANES,), j, jnp.int32) for j in range(p)]

        ins = [in0, in1]
        isems = [si0, si1]
        stages = [st0, st1]
        ssems = [ss0, ss1]

        def shuffle_chunk(inbuf, stage, c):
            @plsc.parallel_loop(0, blk_per_chunk, unroll=1)
            def blk(tt):
                o0 = (c * chunk_h + tt) * nw
                hvec = offh_v[pl.ds(o0, _LANES)]
                wvec = offw_v[pl.ds(o0, _LANES)]
                rvec = jnp.full((_LANES,), tt, jnp.int32)
                for j in range(p):
                    v = plsc.load_gather(inbuf, [hvec, wvec])
                    plsc.store_scatter(stage, [rvec, cvecs[j]], v)
                    wvec = wvec + one

        pltpu.async_copy(x_ref.at[b, c0], in0, si0)
        pltpu.async_copy(x_ref.at[b, c0 + 1], in1, si1)

        def pair(k2, carry):
            for h in range(2):
                k = k2 * 2 + h
                cc = c0 + k
                pltpu.make_async_copy(
                    x_ref.at[b, cc], ins[h], isems[h]).wait()
                for c in range(_CHUNKS):
                    st = stages[c % 2]
                    if c >= 2:
                        pltpu.make_async_copy(
                            st,
                            out_ref.at[b, cc, pl.ds((c - 2) * chunk_h,
                                                    chunk_h)],
                            ssems[c % 2]).wait()
                    shuffle_chunk(ins[h], st, c)
                    pltpu.async_copy(
                        st,
                        out_ref.at[b, cc, pl.ds(c * chunk_h, chunk_h)],
                        ssems[c % 2])
                @pl.when(k2 + 1 < imgs_per_w // 2)
                def _():
                    pltpu.async_copy(x_ref.at[b, cc + 2], ins[h], isems[h])
                for c in (_CHUNKS - 2, _CHUNKS - 1):
                    pltpu.make_async_copy(
                        stages[c % 2],
                        out_ref.at[b, cc, pl.ds(c * chunk_h, chunk_h)],
                        ssems[c % 2]).wait()
            return carry

        lax.fori_loop(0, imgs_per_w // 2, pair, 0)

    f = pl.kernel(
        body,
        out_type=jax.ShapeDtypeStruct((B, C, H, W), jnp.float32),
        mesh=plsc.VectorSubcoreMesh(
            core_axis_name="c", subcore_axis_name="s",
            num_cores=_NC, num_subcores=_NS),
        compiler_params=pltpu.CompilerParams(
            use_tc_tiling_on_sc=True, needs_layout_passes=False),
        scratch_types=[
            pltpu.VMEM((H, W), jnp.float32),
            pltpu.VMEM((H, W), jnp.float32),
            pltpu.VMEM((chunk_h, W), jnp.float32),
            pltpu.VMEM((chunk_h, W), jnp.float32),
            pltpu.VMEM((rows,), jnp.int32),
            pltpu.VMEM((rows,), jnp.int32),
            pltpu.SemaphoreType.DMA,
            pltpu.SemaphoreType.DMA,
            pltpu.SemaphoreType.DMA,
            pltpu.SemaphoreType.DMA,
        ],
    )
    return f(x, off_h, off_w)


# speedup vs baseline: 4.0553x; 1.1877x over previous
import functools

import jax
import jax.numpy as jnp
import numpy as np
from jax import lax
from jax.experimental import pallas as pl
from jax.experimental.pallas import tpu as pltpu
from jax.experimental.pallas import tpu_sc as plsc

_PATCH = 14
_NC, _NS = 2, 16
_NW = _NC * _NS
_LANES = 16
_CHUNKS = 14

_TABLE_CACHE = {}


def _perm_tables(B, nh, nw):
    key_ = (B, nh, nw)
    p = _PATCH
    L = nh * nw
    rows = nh * p * nw

    def make():
        key = jax.random.key(42)
        keys = jax.random.split(key, B)
        return jnp.stack([jax.random.permutation(k, L) for k in keys])

    if key_ not in _TABLE_CACHE:
        try:
            with jax.default_device(jax.devices("cpu")[0]), \
                    jax.ensure_compile_time_eval():
                _TABLE_CACHE[key_] = np.asarray(make())
        except Exception:
            pass

    if key_ in _TABLE_CACHE:
        perms = _TABLE_CACHE[key_]
        xp = np
    else:
        perms = make()
        xp = jnp

    sh = (perms // nw).reshape(B, nh, nw)
    sw = (perms % nw).reshape(B, nh, nw)
    r = xp.arange(p, dtype=xp.int32)
    off_h = sh[:, :, None, :] * p + r[None, None, :, None]
    off_w = xp.broadcast_to(sw[:, :, None, :] * p, (B, nh, p, nw))
    return (off_h.reshape(B, rows).astype(xp.int32),
            off_w.reshape(B, rows).astype(xp.int32))


def kernel(x):
    B, C, H, W = x.shape
    p = _PATCH
    nh, nw = H // p, W // p
    rows = H * nw
    n_img = B * C
    assert n_img % _NW == 0
    imgs_per_w = n_img // _NW
    assert imgs_per_w % 2 == 0
    assert C % imgs_per_w == 0
    assert H % _CHUNKS == 0

    chunk_h = H // _CHUNKS
    blk_per_chunk = chunk_h

    off_h_np, off_w_np = _perm_tables(B, nh, nw)
    off_h = jnp.asarray(off_h_np)
    off_w = jnp.asarray(off_w_np)

    def body(x_ref, offh_ref, offw_ref, out_ref,
             in0, in1, st0, st1, offh_v, offw_v, si0, si1, ss0, ss1):
        cid = lax.axis_index("c")
        sid = lax.axis_index("s")
        wid = sid * _NC + cid
        first = wid * imgs_per_w
        b = first // C
        c0 = first - b * C
        pltpu.sync_copy(offh_ref.at[b], offh_v)
        pltpu.sync_copy(offw_ref.at[b], offw_v)

        iota = lax.iota(jnp.int32, _LANES)
        i14 = iota * jnp.full((_LANES,), p, jnp.int32)
        one = jnp.full((_LANES,), 1, jnp.int32)
        cvecs = [i14 + jnp.full((_LANES,), j, jnp.int32) for j in range(p)]

        ins = [in0, in1]
        isems = [si0, si1]
        stages = [st0, st1]
        ssems = [ss0, ss1]

        def shuffle_chunk(inbuf, stage, c):
            @plsc.parallel_loop(0, blk_per_chunk, unroll=1)
            def blk(tt):
                o0 = (c * chunk_h + tt) * nw
                hvec = offh_v[pl.ds(o0, _LANES)]
                wvec = offw_v[pl.ds(o0, _LANES)]
                rvec = jnp.full((_LANES,), tt, jnp.int32)
                for j in range(p):
                    v = plsc.load_gather(inbuf, [hvec, wvec])
                    plsc.store_scatter(stage, [rvec, cvecs[j]], v)
                    wvec = wvec + one

        pltpu.async_copy(x_ref.at[b, c0], in0, si0)
        pltpu.async_copy(x_ref.at[b, c0 + 1], in1, si1)

        def pair(k2, carry):
            for h in range(2):
                k = k2 * 2 + h
                cc = c0 + k
                pltpu.make_async_copy(
                    x_ref.at[b, cc], ins[h], isems[h]).wait()

                def chunk_pair(q, carry2):
                    for s in range(2):
                        c = q * 2 + s
                        st = stages[s]

                        @pl.when(q > 0)
                        def _():
                            pltpu.make_async_copy(
                                st,
                                out_ref.at[b, cc,
                                           pl.ds((c - 2) * chunk_h,
                                                 chunk_h)],
                                ssems[s]).wait()

                        shuffle_chunk(ins[h], st, c)
                        pltpu.async_copy(
                            st,
                            out_ref.at[b, cc, pl.ds(c * chunk_h, chunk_h)],
                            ssems[s])
                    return carry2

                lax.fori_loop(0, _CHUNKS // 2, chunk_pair, 0)
                @pl.when(k2 + 1 < imgs_per_w // 2)
                def _():
                    pltpu.async_copy(x_ref.at[b, cc + 2], ins[h], isems[h])
                for c in (_CHUNKS - 2, _CHUNKS - 1):
                    pltpu.make_async_copy(
                        stages[c % 2],
                        out_ref.at[b, cc, pl.ds(c * chunk_h, chunk_h)],
                        ssems[c % 2]).wait()
            return carry

        lax.fori_loop(0, imgs_per_w // 2, pair, 0)

    f = pl.kernel(
        body,
        out_type=jax.ShapeDtypeStruct((B, C, H, W), jnp.float32),
        mesh=plsc.VectorSubcoreMesh(
            core_axis_name="c", subcore_axis_name="s",
            num_cores=_NC, num_subcores=_NS),
        compiler_params=pltpu.CompilerParams(
            use_tc_tiling_on_sc=True, needs_layout_passes=False),
        scratch_types=[
            pltpu.VMEM((H, W), jnp.float32),
            pltpu.VMEM((H, W), jnp.float32),
            pltpu.VMEM((chunk_h, W), jnp.float32),
            pltpu.VMEM((chunk_h, W), jnp.float32),
            pltpu.VMEM((rows,), jnp.int32),
            pltpu.VMEM((rows,), jnp.int32),
            pltpu.SemaphoreType.DMA,
            pltpu.SemaphoreType.DMA,
            pltpu.SemaphoreType.DMA,
            pltpu.SemaphoreType.DMA,
        ],
    )
    return f(x, off_h, off_w)


# speedup vs baseline: 4.0570x; 1.0004x over previous
import jax
import jax.numpy as jnp
import numpy as np
from jax import lax
from jax.experimental import pallas as pl
from jax.experimental.pallas import tpu as pltpu
from jax.experimental.pallas import tpu_sc as plsc

_PATCH = 14
_NC, _NS = 2, 16
_NW = _NC * _NS
_LANES = 16
_CHUNKS = 14

_TABLE_CACHE = {}


def _perm_tables(B, nh, nw):
    key_ = (B, nh, nw)
    p = _PATCH
    L = nh * nw
    rows = nh * p * nw

    def make():
        key = jax.random.key(42)
        keys = jax.random.split(key, B)
        return jnp.stack([jax.random.permutation(k, L) for k in keys])

    if key_ not in _TABLE_CACHE:
        try:
            with jax.default_device(jax.devices("cpu")[0]), \
                    jax.ensure_compile_time_eval():
                _TABLE_CACHE[key_] = np.asarray(make())
        except Exception:
            pass

    if key_ in _TABLE_CACHE:
        perms = _TABLE_CACHE[key_]
        xp = np
    else:
        perms = make()
        xp = jnp

    sh = (perms // nw).reshape(B, nh, nw)
    sw = (perms % nw).reshape(B, nh, nw)
    r = xp.arange(p, dtype=xp.int32)
    off_h = sh[:, :, None, :] * p + r[None, None, :, None]
    off_w = xp.broadcast_to(sw[:, :, None, :] * p, (B, nh, p, nw))
    return (off_h.reshape(B, rows).astype(xp.int32),
            off_w.reshape(B, rows).astype(xp.int32))


def kernel(x):
    B, C, H, W = x.shape
    p = _PATCH
    nh, nw = H // p, W // p
    rows = H * nw
    n_img = B * C
    assert n_img % _NW == 0
    imgs_per_w = n_img // _NW
    assert imgs_per_w % 2 == 0
    assert C % imgs_per_w == 0
    assert H % _CHUNKS == 0

    chunk_h = H // _CHUNKS
    blk_per_chunk = chunk_h

    off_h_np, off_w_np = _perm_tables(B, nh, nw)
    off_h = jnp.asarray(off_h_np)
    off_w = jnp.asarray(off_w_np)

    def body(x_ref, offh_ref, offw_ref, out_ref,
             in0, in1, st0, st1, offh_v, offw_v, si0, si1, ss0, ss1):
        cid = lax.axis_index("c")
        sid = lax.axis_index("s")
        wid = sid * _NC + cid
        first = wid * imgs_per_w
        b = first // C
        c0 = first - b * C
        pltpu.sync_copy(offh_ref.at[b], offh_v)
        pltpu.sync_copy(offw_ref.at[b], offw_v)

        iota = lax.iota(jnp.int32, _LANES)
        i14 = iota * jnp.full((_LANES,), p, jnp.int32)
        one = jnp.full((_LANES,), 1, jnp.int32)
        cvecs = [i14 + jnp.full((_LANES,), j, jnp.int32) for j in range(p)]

        ins = [in0, in1]
        isems = [si0, si1]
        stages = [st0, st1]
        ssems = [ss0, ss1]

        def shuffle_chunk(inbuf, stage, c):
            @plsc.parallel_loop(0, blk_per_chunk, unroll=1)
            def blk(tt):
                o0 = (c * chunk_h + tt) * nw
                hvec = offh_v[pl.ds(o0, _LANES)]
                wvec = offw_v[pl.ds(o0, _LANES)]
                rvec = jnp.full((_LANES,), tt, jnp.int32)
                for j in range(p):
                    v = plsc.load_gather(inbuf, [hvec, wvec])
                    plsc.store_scatter(stage, [rvec, cvecs[j]], v)
                    wvec = wvec + one

        pltpu.async_copy(x_ref.at[b, c0], in0, si0)
        pltpu.async_copy(x_ref.at[b, c0 + 1], in1, si1)

        def pair(k2, carry):
            for h in range(2):
                k = k2 * 2 + h
                cc = c0 + k
                pltpu.make_async_copy(
                    x_ref.at[b, cc], ins[h], isems[h]).wait()

                def chunk_pair(q, carry2):
                    for s in range(2):
                        c = q * 2 + s
                        st = stages[s]

                        @pl.when(q > 0)
                        def _():
                            pltpu.make_async_copy(
                                st,
                                out_ref.at[b, cc,
                                           pl.ds((c - 2) * chunk_h,
                                                 chunk_h)],
                                ssems[s]).wait()

                        shuffle_chunk(ins[h], st, c)
                        pltpu.async_copy(
                            st,
                            out_ref.at[b, cc, pl.ds(c * chunk_h, chunk_h)],
                            ssems[s])
                    return carry2

                lax.fori_loop(0, _CHUNKS // 2, chunk_pair, 0)
                @pl.when(k2 + 1 < imgs_per_w // 2)
                def _():
                    pltpu.async_copy(x_ref.at[b, cc + 2], ins[h], isems[h])
                for c in (_CHUNKS - 2, _CHUNKS - 1):
                    pltpu.make_async_copy(
                        stages[c % 2],
                        out_ref.at[b, cc, pl.ds(c * chunk_h, chunk_h)],
                        ssems[c % 2]).wait()
            return carry

        lax.fori_loop(0, imgs_per_w // 2, pair, 0)

    f = pl.kernel(
        body,
        out_type=jax.ShapeDtypeStruct((B, C, H, W), jnp.float32),
        mesh=plsc.VectorSubcoreMesh(
            core_axis_name="c", subcore_axis_name="s",
            num_cores=_NC, num_subcores=_NS),
        compiler_params=pltpu.CompilerParams(
            use_tc_tiling_on_sc=True, needs_layout_passes=False),
        scratch_types=[
            pltpu.VMEM((H, W), jnp.float32),
            pltpu.VMEM((H, W), jnp.float32),
            pltpu.VMEM((chunk_h, W), jnp.float32),
            pltpu.VMEM((chunk_h, W), jnp.float32),
            pltpu.VMEM((rows,), jnp.int32),
            pltpu.VMEM((rows,), jnp.int32),
            pltpu.SemaphoreType.DMA,
            pltpu.SemaphoreType.DMA,
            pltpu.SemaphoreType.DMA,
            pltpu.SemaphoreType.DMA,
        ],
    )
    return f(x, off_h, off_w)
